# baseline (speedup 1.0000x reference)
import jax
import jax.numpy as jnp
from jax.experimental import pallas as pl
from jax.experimental.pallas import tpu as pltpu

G = 200
A = 20000
BA = 4096
NB = 5
Ap = BA * NB
IOU_THRESHOLD = 0.5
BIG = 2**30


def _body(boxes_ref, anch_ref, table_ref, bbox_out, lab_out,
          biou_s, bidx_s, acc_iou, acc_idx, gcol_s, gb_s):
    p = pl.program_id(0)
    j = pl.program_id(1)

    @pl.when((p == 0) & (j == 0))
    def _hoist():
        ones = jnp.ones((G, BA), jnp.float32)
        bl = boxes_ref[:, 0:1] * ones
        bt = boxes_ref[:, 1:2] * ones
        br = boxes_ref[:, 2:3] * ones
        bb = boxes_ref[:, 3:4] * ones
        gcol_s[0] = bl
        gcol_s[1] = bt
        gcol_s[2] = br
        gcol_s[3] = bb
        gcol_s[4] = (br - bl) * (bb - bt)

    @pl.when(p == 0)
    def _stage1():
        bl = gcol_s[0]
        bt = gcol_s[1]
        br = gcol_s[2]
        bb = gcol_s[3]
        a1 = gcol_s[4]
        al = anch_ref[0:1, :]
        at = anch_ref[1:2, :]
        ar = anch_ref[2:3, :]
        ab = anch_ref[3:4, :]

        w = jnp.maximum(jnp.minimum(br, ar) - jnp.maximum(bl, al), 0.0)
        h = jnp.maximum(jnp.minimum(bb, ab) - jnp.maximum(bt, at), 0.0)
        inter = w * h
        a2 = (ar - al) * (ab - at)
        iou = inter / (a1 + a2 - inter)

        gi = jax.lax.broadcasted_iota(jnp.int32, (G, BA), 0)
        ai = jax.lax.broadcasted_iota(jnp.int32, (G, BA), 1) + j * BA

        m = jnp.max(iou, axis=0, keepdims=True)
        amin = jnp.min(jnp.where(iou == m, gi, BIG), axis=0, keepdims=True)
        biou_s[0:1, pl.ds(j * BA, BA)] = m
        bidx_s[0:1, pl.ds(j * BA, BA)] = amin

        rmax = jnp.max(iou, axis=1, keepdims=True)
        ridx = jnp.min(jnp.where(iou == rmax, ai, BIG), axis=1, keepdims=True)

        @pl.when(j == 0)
        def _():
            acc_iou[:, 0:1] = jnp.full((G, 1), -1.0, jnp.float32)

        prev_i = acc_iou[:, 0:1]
        upd = rmax > prev_i
        acc_iou[:, 0:1] = jnp.where(upd, rmax, prev_i)

        @pl.when(j == 0)
        def _():
            acc_idx[:, 0:1] = ridx

        @pl.when(j > 0)
        def _():
            acc_idx[:, 0:1] = jnp.where(upd, ridx, acc_idx[:, 0:1])

        @pl.when(j == NB - 1)
        def _():
            gb_s[:, :] = acc_idx[:, 0:1] * jnp.ones((G, BA), jnp.int32)

    @pl.when(p == 1)
    def _stage2():
        ai = jax.lax.broadcasted_iota(jnp.int32, (G, BA), 1) + j * BA
        gi = jax.lax.broadcasted_iota(jnp.int32, (G, BA), 0)

        eqf = gb_s[:, :] == ai
        forced_g = jnp.max(jnp.where(eqf, gi, -1), axis=0, keepdims=True)
        forced = forced_g >= 0
        bidx = bidx_s[0:1, pl.ds(j * BA, BA)]
        biou = biou_s[0:1, pl.ds(j * BA, BA)]
        final_g = jnp.where(forced, forced_g, bidx)
        mask = forced | (biou > IOU_THRESHOLD)

        onehot = (gi == final_g).astype(jnp.float32)
        gath = jax.lax.dot_general(
            table_ref[:, :], onehot, (((1,), (0,)), ((), ())),
            preferred_element_type=jnp.float32,
            precision=jax.lax.Precision.HIGHEST)

        al = anch_ref[0:1, :]
        at = anch_ref[1:2, :]
        ar = anch_ref[2:3, :]
        ab = anch_ref[3:4, :]
        L = jnp.where(mask, gath[0:1, :], al)
        T = jnp.where(mask, gath[1:2, :], at)
        R = jnp.where(mask, gath[2:3, :], ar)
        B = jnp.where(mask, gath[3:4, :], ab)
        bbox_out[0:1, :] = 0.5 * (L + R)
        bbox_out[1:2, :] = 0.5 * (T + B)
        bbox_out[2:3, :] = R - L
        bbox_out[3:4, :] = B - T
        lab = jnp.floor(gath[4:5, :] + 0.5).astype(jnp.int32)
        lab_out[0:1, :] = jnp.where(mask, lab, 0)


@jax.jit
def _run(image, boxes, labels, anchors):
    f32 = jnp.float32
    boxes = boxes.astype(f32)
    anchors = anchors.astype(f32)
    anch_t = jnp.zeros((4, Ap), f32).at[:, :A].set(anchors.T)
    table_t = (jnp.zeros((8, G), f32)
               .at[0:4, :].set(boxes.T)
               .at[4, :].set(labels.astype(f32)))

    bbox_t, lab = pl.pallas_call(
        _body,
        grid=(2, NB),
        in_specs=[
            pl.BlockSpec((G, 4), lambda p, j: (0, 0)),
            pl.BlockSpec((4, BA), lambda p, j: (0, j)),
            pl.BlockSpec((8, G), lambda p, j: (0, 0)),
        ],
        out_specs=[
            pl.BlockSpec((4, BA), lambda p, j: (0, j)),
            pl.BlockSpec((1, BA), lambda p, j: (0, j)),
        ],
        out_shape=[
            jax.ShapeDtypeStruct((4, Ap), f32),
            jax.ShapeDtypeStruct((1, Ap), jnp.int32),
        ],
        scratch_shapes=[
            pltpu.VMEM((1, Ap), f32),
            pltpu.VMEM((1, Ap), jnp.int32),
            pltpu.VMEM((G, 128), f32),
            pltpu.VMEM((G, 128), jnp.int32),
            pltpu.VMEM((5, G, BA), f32),
            pltpu.VMEM((G, BA), jnp.int32),
        ],
    )(boxes, anch_t, table_t)

    bboxes_out = bbox_t[:, :A].T
    labels_out = lab[0, :A]
    return (image, bboxes_out, labels_out)


def kernel(image, boxes, labels, anchors):
    return _run(image, boxes, labels, anchors)
